# baseline (speedup 1.0000x reference)
import functools

import jax
import jax.numpy as jnp
from jax import lax
from jax.experimental import pallas as pl
from jax.experimental.pallas import tpu as pltpu
from jax.experimental.pallas import tpu_sc as plsc

EMB = 64
ROWS = 320
NB = 4


@functools.partial(jax.jit, static_argnums=(2, 3))
def _sc_embedding_gather(idx3, table, num_workers, nchunk):
    mesh = plsc.VectorSubcoreMesh(core_axis_name="c", subcore_axis_name="s")
    total_rows = num_workers * nchunk * ROWS

    @functools.partial(
        pl.kernel,
        mesh=mesh,
        out_type=jax.ShapeDtypeStruct((total_rows, EMB), jnp.float32),
        scratch_types=[
            pltpu.VMEM((nchunk, ROWS), jnp.int32),
        ]
        + [pltpu.VMEM((ROWS, EMB), jnp.float32)] * NB
        + [pltpu.SemaphoreType.DMA] * (2 * NB),
        compiler_params=pltpu.CompilerParams(use_tc_tiling_on_sc=False),
    )
    def k(idx_hbm, table_hbm, out_hbm, idx_v, *bufs_sems):
        bufs = bufs_sems[:NB]
        sgs = bufs_sems[NB : 2 * NB]
        sws = bufs_sems[2 * NB :]
        num_cores = lax.axis_size("c")
        wid = lax.axis_index("s") * num_cores + lax.axis_index("c")
        pltpu.sync_copy(idx_hbm.at[wid], idx_v)
        base = wid * nchunk * ROWS

        def fire(c, r):
            pltpu.async_copy(table_hbm.at[idx_v.at[c]], bufs[r], sgs[r])

        def drain(sem, ref):
            pltpu.make_async_copy(out_hbm.at[pl.ds(0, ref.shape[0])], ref, sem).wait()

        fire(0, 0)
        fire(1, 1)

        def body(g, carry):
            for b in range(NB):
                c = NB * g + b
                r2 = (b + 2) % NB

                @pl.when(c >= 2)
                def _():
                    drain(sws[r2], bufs[r2])

                @pl.when(c + 2 < nchunk)
                def _():
                    fire(c + 2, r2)

                drain(sgs[b], bufs[b])

                pltpu.async_copy(
                    bufs[b],
                    out_hbm.at[pl.ds(base + c * ROWS, ROWS)],
                    sws[b],
                )
            return carry

        lax.fori_loop(0, nchunk // NB, body, 0)
        drain(sws[(nchunk - 2) % NB], bufs[(nchunk - 2) % NB])
        drain(sws[(nchunk - 1) % NB], bufs[(nchunk - 1) % NB])

    return k(idx3, table)


def kernel(indices, table):
    batch, hist = indices.shape
    total = batch * hist
    num_workers = 32
    assert total % (num_workers * ROWS * NB) == 0
    nchunk = total // (num_workers * ROWS)
    idx3 = indices.reshape(num_workers, nchunk, ROWS)
    out = _sc_embedding_gather(idx3, table, num_workers, nchunk)
    return out.reshape(batch, hist, EMB)


# speedup vs baseline: 1.0022x; 1.0022x over previous
import functools

import jax
import jax.numpy as jnp
from jax import lax
from jax.experimental import pallas as pl
from jax.experimental.pallas import tpu as pltpu
from jax.experimental.pallas import tpu_sc as plsc

EMB = 64
ROWS = 512
NUM_WORKERS = 32


@functools.partial(jax.jit, static_argnums=(2,))
def _sc_embedding_gather(idx3, table, nchunk):
    mesh = plsc.VectorSubcoreMesh(core_axis_name="c", subcore_axis_name="s")
    total_rows = NUM_WORKERS * nchunk * ROWS

    @functools.partial(
        pl.kernel,
        mesh=mesh,
        out_type=jax.ShapeDtypeStruct((total_rows, EMB), jnp.float32),
        scratch_types=[
            pltpu.VMEM((nchunk, ROWS), jnp.int32),
            pltpu.VMEM((ROWS, EMB), jnp.float32),
            pltpu.VMEM((ROWS, EMB), jnp.float32),
            pltpu.SemaphoreType.DMA,
            pltpu.SemaphoreType.DMA,
            pltpu.SemaphoreType.DMA,
            pltpu.SemaphoreType.DMA,
        ],
        compiler_params=pltpu.CompilerParams(use_tc_tiling_on_sc=False),
    )
    def k(idx_hbm, table_hbm, out_hbm, idx_v, buf0, buf1, sg0, sg1, sw0, sw1):
        num_cores = lax.axis_size("c")
        wid = lax.axis_index("s") * num_cores + lax.axis_index("c")
        pltpu.sync_copy(idx_hbm.at[wid], idx_v)
        base = wid * nchunk * ROWS
        bufs = (buf0, buf1)
        sgs = (sg0, sg1)
        sws = (sw0, sw1)

        def fire(c, buf, sem):
            pltpu.async_copy(table_hbm.at[idx_v.at[c]], buf, sem)

        def drain(sem, ref):
            pltpu.make_async_copy(out_hbm.at[pl.ds(0, ref.shape[0])], ref, sem).wait()

        fire(0, buf0, sg0)

        def body(g, carry):
            for b in range(2):
                c = 2 * g + b
                nb = 1 - b

                @pl.when(c >= 1)
                def _():
                    drain(sws[nb], bufs[nb])

                @pl.when(c + 1 < nchunk)
                def _():
                    fire(c + 1, bufs[nb], sgs[nb])

                drain(sgs[b], bufs[b])

                pltpu.async_copy(
                    bufs[b],
                    out_hbm.at[pl.ds(base + c * ROWS, ROWS)],
                    sws[b],
                )
            return carry

        lax.fori_loop(0, nchunk // 2, body, 0)
        drain(sws[1], buf1)

    return k(idx3, table)


def kernel(indices, table):
    batch, hist = indices.shape
    total = batch * hist
    assert total % (NUM_WORKERS * ROWS * 2) == 0
    nchunk = total // (NUM_WORKERS * ROWS)
    idx3 = indices.reshape(NUM_WORKERS, nchunk, ROWS)
    out = _sc_embedding_gather(idx3, table, nchunk)
    return out.reshape(batch, hist, EMB)
